# baseline (speedup 1.0000x reference)
import functools

import jax
import jax.numpy as jnp
from jax.experimental import pallas as pl
from jax.experimental.pallas import tpu as pltpu

N = 4096
F = 128

BI = 1024
BR = 512


def _prep_kernel(a_ref, x_ref, w0_ref, dis_ref, a2_ref, hws1t_ref):
    i = pl.program_id(0)
    a = a_ref[...]
    a2_ref[...] = a.astype(jnp.int4)
    s = jnp.sum(a.astype(jnp.float32), axis=0, keepdims=True)

    @pl.when(i == 0)
    def _():
        dis_ref[...] = s

    @pl.when(i > 0)
    def _():
        dis_ref[...] += s

    @pl.when(i == pl.num_programs(0) - 1)
    def _():
        dis = jax.lax.rsqrt(dis_ref[...] + 1.0)
        dis_ref[...] = dis
        hws1t_ref[...] = (
            jax.lax.dot_general(
                w0_ref[...], x_ref[...], (((0,), (1,)), ((), ())),
                preferred_element_type=jnp.float32,
            )
            * dis
        ).astype(jnp.bfloat16)


def _gcn_kernel(a2_ref, hws1t_ref, w1_ref, b_ref, dis_ref, o_ref, hwst_ref,
                h1t_ref):
    l = pl.program_id(0)
    i = pl.program_id(1)

    @pl.when((l == 0) & (i == 0))
    def _():
        hwst_ref[...] = hws1t_ref[...]

    @pl.when((l == 1) & (i == 0))
    def _():
        hwst_ref[...] = (
            jax.lax.dot_general(
                w1_ref[...], h1t_ref[...], (((0,), (0,)), ((), ())),
                preferred_element_type=jnp.float32,
            )
            * dis_ref[...]
        ).astype(jnp.bfloat16)

    a = a2_ref[...].astype(jnp.bfloat16)
    acc = jax.lax.dot_general(
        hwst_ref[...], a, (((1,), (0,)), ((), ())),
        preferred_element_type=jnp.float32,
    )
    sl = pl.ds(i * BI, BI)
    di = dis_ref[:, sl]
    res = jnp.maximum(
        di * (acc + hwst_ref[:, sl].astype(jnp.float32)) + b_ref[0], 0.0
    )

    @pl.when(l == 0)
    def _():
        h1t_ref[:, sl] = res

    o_ref[0] = res


def kernel(features, A, W0, b0, W1, b1):
    dis_row, a2, hws1t = pl.pallas_call(
        _prep_kernel,
        grid=(N // BR,),
        in_specs=[
            pl.BlockSpec((BR, N), lambda i: (i, 0)),
            pl.BlockSpec((N, F), lambda i: (0, 0)),
            pl.BlockSpec((F, F), lambda i: (0, 0)),
        ],
        out_specs=[
            pl.BlockSpec((1, N), lambda i: (0, 0)),
            pl.BlockSpec((BR, N), lambda i: (i, 0)),
            pl.BlockSpec((F, N), lambda i: (0, 0)),
        ],
        out_shape=[
            jax.ShapeDtypeStruct((1, N), jnp.float32),
            jax.ShapeDtypeStruct((N, N), jnp.int4),
            jax.ShapeDtypeStruct((F, N), jnp.bfloat16),
        ],
    )(A, features, W0)

    b_stack = jnp.stack([b0.reshape(F, 1), b1.reshape(F, 1)])

    out = pl.pallas_call(
        _gcn_kernel,
        grid=(2, N // BI),
        in_specs=[
            pl.BlockSpec((N, BI), lambda l, i: (0, i)),
            pl.BlockSpec((F, N), lambda l, i: (0, 0)),
            pl.BlockSpec((F, F), lambda l, i: (0, 0)),
            pl.BlockSpec((1, F, 1), lambda l, i: (l, 0, 0)),
            pl.BlockSpec((1, N), lambda l, i: (0, 0)),
        ],
        out_specs=pl.BlockSpec((1, F, BI), lambda l, i: (l, 0, i)),
        out_shape=jax.ShapeDtypeStruct((2, F, N), jnp.float32),
        scratch_shapes=[
            pltpu.VMEM((F, N), jnp.bfloat16),
            pltpu.VMEM((F, N), jnp.float32),
        ],
        compiler_params=pltpu.CompilerParams(
            dimension_semantics=("arbitrary", "arbitrary"),
        ),
    )(a2, hws1t, W1, b_stack, dis_row)
    return out[1].T


# speedup vs baseline: 1.0442x; 1.0442x over previous
import functools

import jax
import jax.numpy as jnp
from jax.experimental import pallas as pl
from jax.experimental.pallas import tpu as pltpu

N = 4096
F = 128

BI = 1024
BR = 512


def _prep_kernel(a_ref, x_ref, w0_ref, dis_ref, a2_ref, hws1t_ref):
    i = pl.program_id(0)
    a = a_ref[...]
    a2_ref[...] = a.astype(jnp.int2)
    s = jnp.sum(a.astype(jnp.float32), axis=0, keepdims=True)

    @pl.when(i == 0)
    def _():
        dis_ref[...] = s

    @pl.when(i > 0)
    def _():
        dis_ref[...] += s

    @pl.when(i == pl.num_programs(0) - 1)
    def _():
        dis = jax.lax.rsqrt(dis_ref[...] + 1.0)
        dis_ref[...] = dis
        hws1t_ref[...] = (
            jax.lax.dot_general(
                w0_ref[...], x_ref[...], (((0,), (1,)), ((), ())),
                preferred_element_type=jnp.float32,
            )
            * dis
        ).astype(jnp.bfloat16)


def _gcn_kernel(a2_ref, hws1t_ref, w1_ref, b_ref, dis_ref, o_ref, hwst_ref,
                h1t_ref):
    l = pl.program_id(0)
    i = pl.program_id(1)

    @pl.when((l == 0) & (i == 0))
    def _():
        hwst_ref[...] = hws1t_ref[...]

    @pl.when((l == 1) & (i == 0))
    def _():
        hwst_ref[...] = (
            jax.lax.dot_general(
                w1_ref[...], h1t_ref[...], (((0,), (0,)), ((), ())),
                preferred_element_type=jnp.float32,
            )
            * dis_ref[...]
        ).astype(jnp.bfloat16)

    a = a2_ref[...].astype(jnp.bfloat16)
    acc = jax.lax.dot_general(
        hwst_ref[...], a, (((1,), (0,)), ((), ())),
        preferred_element_type=jnp.float32,
    )
    sl = pl.ds(i * BI, BI)
    di = dis_ref[:, sl]
    res = jnp.maximum(
        di * (acc + hwst_ref[:, sl].astype(jnp.float32)) + b_ref[0], 0.0
    )

    @pl.when(l == 0)
    def _():
        h1t_ref[:, sl] = res

    o_ref[0] = res.T


def kernel(features, A, W0, b0, W1, b1):
    dis_row, a2, hws1t = pl.pallas_call(
        _prep_kernel,
        grid=(N // BR,),
        in_specs=[
            pl.BlockSpec((BR, N), lambda i: (i, 0)),
            pl.BlockSpec((N, F), lambda i: (0, 0)),
            pl.BlockSpec((F, F), lambda i: (0, 0)),
        ],
        out_specs=[
            pl.BlockSpec((1, N), lambda i: (0, 0)),
            pl.BlockSpec((BR, N), lambda i: (i, 0)),
            pl.BlockSpec((F, N), lambda i: (0, 0)),
        ],
        out_shape=[
            jax.ShapeDtypeStruct((1, N), jnp.float32),
            jax.ShapeDtypeStruct((N, N), jnp.int2),
            jax.ShapeDtypeStruct((F, N), jnp.bfloat16),
        ],
    )(A, features, W0)

    b_stack = jnp.stack([b0.reshape(F, 1), b1.reshape(F, 1)])

    out = pl.pallas_call(
        _gcn_kernel,
        grid=(2, N // BI),
        in_specs=[
            pl.BlockSpec((N, BI), lambda l, i: (0, i)),
            pl.BlockSpec((F, N), lambda l, i: (0, 0)),
            pl.BlockSpec((F, F), lambda l, i: (0, 0)),
            pl.BlockSpec((1, F, 1), lambda l, i: (l, 0, 0)),
            pl.BlockSpec((1, N), lambda l, i: (0, 0)),
        ],
        out_specs=pl.BlockSpec((1, BI, F), lambda l, i: (l, i, 0)),
        out_shape=jax.ShapeDtypeStruct((2, N, F), jnp.float32),
        scratch_shapes=[
            pltpu.VMEM((F, N), jnp.bfloat16),
            pltpu.VMEM((F, N), jnp.float32),
        ],
        compiler_params=pltpu.CompilerParams(
            dimension_semantics=("arbitrary", "arbitrary"),
        ),
    )(a2, hws1t, W1, b_stack, dis_row)
    return out[1]


# speedup vs baseline: 1.0742x; 1.0287x over previous
import functools

import jax
import jax.numpy as jnp
from jax.experimental import pallas as pl
from jax.experimental.pallas import tpu as pltpu

N = 4096
F = 128

BI = 2048
BR = 512


def _prep_kernel(a_ref, x_ref, w0_ref, dis_ref, a2_ref, hws1t_ref):
    i = pl.program_id(0)
    a = a_ref[...]
    a2_ref[...] = a.astype(jnp.int2)
    s = jnp.sum(a.astype(jnp.float32), axis=0, keepdims=True)

    @pl.when(i == 0)
    def _():
        dis_ref[...] = s

    @pl.when(i > 0)
    def _():
        dis_ref[...] += s

    @pl.when(i == pl.num_programs(0) - 1)
    def _():
        dis = jax.lax.rsqrt(dis_ref[...] + 1.0)
        dis_ref[...] = dis
        hws1t_ref[...] = (
            jax.lax.dot_general(
                w0_ref[...], x_ref[...], (((0,), (1,)), ((), ())),
                preferred_element_type=jnp.float32,
            )
            * dis
        ).astype(jnp.bfloat16)


def _gcn_kernel(a2_ref, hws1t_ref, w1_ref, b_ref, dis_ref, o_ref, hwst_ref,
                h1t_ref):
    l = pl.program_id(0)
    i = pl.program_id(1)

    @pl.when((l == 0) & (i == 0))
    def _():
        hwst_ref[...] = hws1t_ref[...]

    @pl.when((l == 1) & (i == 0))
    def _():
        hwst_ref[...] = (
            jax.lax.dot_general(
                w1_ref[...], h1t_ref[...], (((0,), (0,)), ((), ())),
                preferred_element_type=jnp.float32,
            )
            * dis_ref[...]
        ).astype(jnp.bfloat16)

    a = a2_ref[...].astype(jnp.bfloat16)
    acc = jax.lax.dot_general(
        hwst_ref[...], a, (((1,), (0,)), ((), ())),
        preferred_element_type=jnp.float32,
    )
    sl = pl.ds(i * BI, BI)
    di = dis_ref[:, sl]
    res = jnp.maximum(
        di * (acc + hwst_ref[:, sl].astype(jnp.float32)) + b_ref[0], 0.0
    )

    @pl.when(l == 0)
    def _():
        h1t_ref[:, sl] = res

    o_ref[0] = res


def kernel(features, A, W0, b0, W1, b1):
    dis_row, a2, hws1t = pl.pallas_call(
        _prep_kernel,
        grid=(N // BR,),
        in_specs=[
            pl.BlockSpec((BR, N), lambda i: (i, 0)),
            pl.BlockSpec((N, F), lambda i: (0, 0)),
            pl.BlockSpec((F, F), lambda i: (0, 0)),
        ],
        out_specs=[
            pl.BlockSpec((1, N), lambda i: (0, 0)),
            pl.BlockSpec((BR, N), lambda i: (i, 0)),
            pl.BlockSpec((F, N), lambda i: (0, 0)),
        ],
        out_shape=[
            jax.ShapeDtypeStruct((1, N), jnp.float32),
            jax.ShapeDtypeStruct((N, N), jnp.int2),
            jax.ShapeDtypeStruct((F, N), jnp.bfloat16),
        ],
    )(A, features, W0)

    b_stack = jnp.stack([b0.reshape(F, 1), b1.reshape(F, 1)])

    out = pl.pallas_call(
        _gcn_kernel,
        grid=(2, N // BI),
        in_specs=[
            pl.BlockSpec((N, BI), lambda l, i: (0, i)),
            pl.BlockSpec((F, N), lambda l, i: (0, 0)),
            pl.BlockSpec((F, F), lambda l, i: (0, 0)),
            pl.BlockSpec((1, F, 1), lambda l, i: (l, 0, 0)),
            pl.BlockSpec((1, N), lambda l, i: (0, 0)),
        ],
        out_specs=pl.BlockSpec((1, F, BI), lambda l, i: (l, 0, i)),
        out_shape=jax.ShapeDtypeStruct((2, F, N), jnp.float32),
        scratch_shapes=[
            pltpu.VMEM((F, N), jnp.bfloat16),
            pltpu.VMEM((F, N), jnp.float32),
        ],
        compiler_params=pltpu.CompilerParams(
            dimension_semantics=("arbitrary", "arbitrary"),
        ),
    )(a2, hws1t, W1, b_stack, dis_row)
    return out[1].T
